# baseline (speedup 1.0000x reference)
import jax
import jax.numpy as jnp
from jax.experimental import pallas as pl

_KNN = 5
_HIGH = jax.lax.Precision.HIGHEST


def _dot(a, b, dims):
    return jax.lax.dot_general(
        a, b, (dims, ((), ())),
        preferred_element_type=jnp.float32, precision=_HIGH)


def _stage1_body(xn_ref, e_ref, w0_ref, g0_ref, b0_ref, anc_ref, sigp_ref,
                 nodes_ref):
    B, N, C = xn_ref.shape

    ones_n = jnp.ones((1, N), jnp.float32)
    hs = []
    ssum = jnp.zeros((1, C), jnp.float32)
    qsum = jnp.zeros((1, C), jnp.float32)
    for b in range(B):
        x1 = jax.nn.sigmoid(e_ref[b]) * xn_ref[b]
        h = _dot(x1, w0_ref[...], ((1,), (1,)))
        hs.append(h)
        ssum = ssum + _dot(ones_n, h, ((1,), (0,)))
        qsum = qsum + _dot(ones_n, h * h, ((1,), (0,)))
    mean = ssum / (B * N)
    var = qsum / (B * N) - mean * mean
    scale = g0_ref[...] / jnp.sqrt(var + 1e-5)
    shift = b0_ref[...] - mean * scale

    sig = jax.nn.sigmoid(sigp_ref[...])
    inv2 = 1.0 / (sig * sig)
    anc = anc_ref[...]
    a1 = anc * inv2
    ones_c = jnp.ones((C, 1), jnp.float32)
    c0 = _dot(anc * a1, ones_c, ((1,), (0,)))
    ones_col = jnp.ones((N, 1), jnp.float32)

    for b in range(B):
        hn = jnp.maximum(hs[b] * scale + shift, 0.0)
        t1 = _dot(inv2, hn * hn, ((1,), (1,)))
        t2 = _dot(a1, hn, ((1,), (1,)))
        logits = -0.5 * t1 + t2 - 0.5 * c0
        m = jnp.max(logits, axis=0, keepdims=True)
        e = jnp.exp(logits - m)
        sa = e / jnp.sum(e, axis=0, keepdims=True)
        den = _dot(sa, ones_col, ((1,), (0,)))
        sxh = _dot(sa, hn, ((1,), (0,)))
        nodes = (sxh - anc * den) / sig / (den + 1e-9)
        rn = jnp.sqrt(jnp.sum(nodes * nodes, axis=1, keepdims=True))
        nodes = nodes / jnp.maximum(rn, 1e-12)
        fl = jnp.sqrt(jnp.sum(nodes * nodes, keepdims=True))
        nodes = nodes / jnp.maximum(fl, 1e-12)
        nodes_ref[b] = nodes


def _rank_body(xc_ref, m1_ref, li_ref):
    B, C, N = xc_ref.shape
    K = m1_ref.shape[2]

    ones_c = jnp.ones((1, C), jnp.float32)
    ones_c_col = jnp.ones((C, 1), jnp.float32)
    for b in range(B):
        xc = xc_ref[b]
        m1 = m1_ref[b]
        mv = _dot(m1, xc, ((0,), (0,)))
        xsq = _dot(ones_c, xc * xc, ((1,), (0,)))
        vsq = _dot(m1 * m1, ones_c_col, ((0,), (0,)))
        d2 = xsq - 2.0 * mv + vsq
        kio = jax.lax.broadcasted_iota(jnp.int32, (K, N), 0).astype(jnp.float32)
        rio = jax.lax.broadcasted_iota(jnp.int32, (_KNN, N), 0).astype(jnp.float32)

        dwork = d2
        limat = jnp.zeros((_KNN, N), jnp.float32)
        for r in range(_KNN):
            mn = jnp.min(dwork, axis=0, keepdims=True)
            li = jnp.min(jnp.where(dwork <= mn, kio, float(K)), axis=0,
                         keepdims=True)
            limat = jnp.where(rio == float(r), li, limat)
            dwork = jnp.where(kio == li, jnp.inf, dwork)
        li_ref[b] = limat


def _edgeconv_body(xn_ref, m1_ref, w1a_ref, wd_ref, c0_ref, c1_ref, c2_ref,
                   c3_ref, c4_ref, g1_ref, b1_ref, out_ref):
    B, N, C = xn_ref.shape
    K = m1_ref.shape[2]
    cols = [c0_ref, c1_ref, c2_ref, c3_ref, c4_ref]

    ones_n = jnp.ones((1, N), jnp.float32)
    ones_k = jnp.ones((1, K), jnp.float32)
    lane = jax.lax.broadcasted_iota(jnp.int32, (N, K), 1).astype(jnp.float32)

    s1 = jnp.zeros((1, C), jnp.float32)
    s2 = jnp.zeros((1, C), jnp.float32)
    for b in range(B):
        q = _dot(xn_ref[b], wd_ref[...], ((1,), (1,)))
        pm = _dot(w1a_ref[...], m1_ref[b], ((1,), (0,)))
        pmt = _dot(m1_ref[b], w1a_ref[...], ((0,), (1,)))
        mfall = jnp.zeros((N, K), jnp.float32)
        for m in range(_KNN):
            mfall = mfall + (lane == cols[m][b]).astype(jnp.float32)
        cnt = _dot(ones_n, mfall, ((1,), (0,)))
        u = _dot(mfall, q, ((0,), (0,)))
        s1 = s1 + _dot(cnt, pm, ((1,), (1,))) \
            + _KNN * _dot(ones_n, q, ((1,), (0,)))
        s2 = s2 + _dot(cnt, pm * pm, ((1,), (1,))) \
            + 2.0 * _dot(ones_k, pmt * u, ((1,), (0,))) \
            + _KNN * _dot(ones_n, q * q, ((1,), (0,)))

    count = B * N * _KNN
    mean = s1 / count
    var = s2 / count - mean * mean
    a = g1_ref[...] / jnp.sqrt(var + 1e-5)
    bb = b1_ref[...] - mean * a
    sgn = jnp.where(a >= 0.0, 1.0, -1.0)

    for b in range(B):
        q = _dot(xn_ref[b], wd_ref[...], ((1,), (1,)))
        pm = _dot(w1a_ref[...], m1_ref[b], ((1,), (0,)))
        smax = jnp.full((N, C), -jnp.inf, jnp.float32)
        for m in range(_KNN):
            mf = (lane == cols[m][b]).astype(jnp.float32)
            g = _dot(mf, pm, ((1,), (1,)))
            smax = jnp.maximum(smax, g * sgn)
        meff = sgn * smax
        y = jnp.maximum(a * (meff + q) + bb, 0.0)
        out_ref[b] = xn_ref[b] + y


def _run(interpret=False):
    def go(xn, xc, en, w0, g0, b0, anc, sigp, w1a, wd, g1, b1):
        B, N, C = xn.shape
        K = anc.shape[0]
        nodes = pl.pallas_call(
            _stage1_body,
            out_shape=jax.ShapeDtypeStruct((B, K, C), jnp.float32),
            interpret=interpret,
        )(xn, en, w0, g0, b0, anc, sigp)
        m1 = nodes.reshape(B, C, K)

        li = pl.pallas_call(
            _rank_body,
            out_shape=jax.ShapeDtypeStruct((B, _KNN, N), jnp.float32),
            interpret=interpret,
        )(xc, m1)
        ids = li.reshape(B, N, _KNN)
        cols = [ids[:, :, m].reshape(B, N, 1) for m in range(_KNN)]

        outn = pl.pallas_call(
            _edgeconv_body,
            out_shape=jax.ShapeDtypeStruct((B, N, C), jnp.float32),
            interpret=interpret,
        )(xn, m1, w1a, wd, *cols, g1, b1)
        return outn
    return go


def kernel(x, edge, W0, gamma0, beta0, anchor, sigma_p, W1, gamma1, beta1):
    B, C, H, W = x.shape
    N = H * W
    xc = x.reshape(B, C, N)
    xn = xc.transpose(0, 2, 1)
    en = edge.reshape(B, N, 1)
    w1a = W1[:, :C]
    wd = W1[:, C:] - w1a
    outn = _run()(xn, xc, en, W0, gamma0.reshape(1, C), beta0.reshape(1, C),
                  anchor, sigma_p, w1a, wd,
                  gamma1.reshape(1, C), beta1.reshape(1, C))
    return outn.transpose(0, 2, 1).reshape(B, C, H, W)


# speedup vs baseline: 1.2578x; 1.2578x over previous
import jax
import jax.numpy as jnp
from jax.experimental import pallas as pl

_KNN = 5
_HIGH = jax.lax.Precision.HIGHEST


def _dot(a, b, dims):
    return jax.lax.dot_general(
        a, b, (dims, ((), ())),
        preferred_element_type=jnp.float32, precision=_HIGH)


def _stage1_body(xn_ref, e_ref, w0_ref, g0_ref, b0_ref, anc_ref, sigp_ref,
                 nodes_ref):
    B, N, C = xn_ref.shape

    hs = []
    ssum = jnp.zeros((1, C), jnp.float32)
    qsum = jnp.zeros((1, C), jnp.float32)
    for b in range(B):
        x1 = jax.nn.sigmoid(e_ref[b]) * xn_ref[b]
        h = _dot(x1, w0_ref[...], ((1,), (1,)))
        hs.append(h)
        ssum = ssum + jnp.sum(h, axis=0, keepdims=True)
        qsum = qsum + jnp.sum(h * h, axis=0, keepdims=True)
    mean = ssum / (B * N)
    var = qsum / (B * N) - mean * mean
    scale = g0_ref[...] / jnp.sqrt(var + 1e-5)
    shift = b0_ref[...] - mean * scale

    sig = jax.nn.sigmoid(sigp_ref[...])
    inv2 = 1.0 / (sig * sig)
    anc = anc_ref[...]
    a1 = anc * inv2
    ones_c = jnp.ones((C, 1), jnp.float32)
    c0 = _dot(anc * a1, ones_c, ((1,), (0,)))
    ones_col = jnp.ones((N, 1), jnp.float32)

    for b in range(B):
        hn = jnp.maximum(hs[b] * scale + shift, 0.0)
        t1 = _dot(inv2, hn * hn, ((1,), (1,)))
        t2 = _dot(a1, hn, ((1,), (1,)))
        logits = -0.5 * t1 + t2 - 0.5 * c0
        m = jnp.max(logits, axis=0, keepdims=True)
        e = jnp.exp(logits - m)
        sa = e / jnp.sum(e, axis=0, keepdims=True)
        den = _dot(sa, ones_col, ((1,), (0,)))
        sxh = _dot(sa, hn, ((1,), (0,)))
        nodes = (sxh - anc * den) / sig / (den + 1e-9)
        rn = jnp.sqrt(jnp.sum(nodes * nodes, axis=1, keepdims=True))
        nodes = nodes / jnp.maximum(rn, 1e-12)
        fl = jnp.sqrt(jnp.sum(nodes * nodes, keepdims=True))
        nodes = nodes / jnp.maximum(fl, 1e-12)
        nodes_ref[b] = nodes


def _rank_body(xc_ref, m1_ref, li_ref):
    B, C, N = xc_ref.shape
    K = m1_ref.shape[2]

    ones_c = jnp.ones((1, C), jnp.float32)
    ones_c_col = jnp.ones((C, 1), jnp.float32)
    for b in range(B):
        xc = xc_ref[b]
        m1 = m1_ref[b]
        mv = _dot(m1, xc, ((0,), (0,)))
        xsq = _dot(ones_c, xc * xc, ((1,), (0,)))
        vsq = _dot(m1 * m1, ones_c_col, ((0,), (0,)))
        d2 = xsq - 2.0 * mv + vsq
        kio = jax.lax.broadcasted_iota(jnp.int32, (K, N), 0).astype(jnp.float32)
        rio = jax.lax.broadcasted_iota(jnp.int32, (_KNN, N), 0).astype(jnp.float32)

        dwork = d2
        limat = jnp.zeros((_KNN, N), jnp.float32)
        for r in range(_KNN):
            mn = jnp.min(dwork, axis=0, keepdims=True)
            li = jnp.min(jnp.where(dwork <= mn, kio, float(K)), axis=0,
                         keepdims=True)
            limat = jnp.where(rio == float(r), li, limat)
            dwork = jnp.where(kio == li, jnp.inf, dwork)
        li_ref[b] = limat


def _edgeconv_body(xn_ref, m1_ref, w1a_ref, wd_ref, ids_ref, g1_ref, b1_ref,
                   out_ref):
    B, N, C = xn_ref.shape
    K = m1_ref.shape[2]

    lane = jax.lax.broadcasted_iota(jnp.int32, (N, K), 1).astype(jnp.float32)
    lane5 = jax.lax.broadcasted_iota(jnp.int32, (N, _KNN), 1).astype(jnp.float32)

    qs, sums, mxs, mns = [], [], [], []
    s1 = jnp.zeros((1, C), jnp.float32)
    s2 = jnp.zeros((1, C), jnp.float32)
    for b in range(B):
        q = _dot(xn_ref[b], wd_ref[...], ((1,), (1,)))
        pm = _dot(w1a_ref[...], m1_ref[b], ((1,), (0,)))
        qs.append(q)
        ids = ids_ref[b]

        ssum = jnp.zeros((N, C), jnp.float32)
        smax = jnp.full((N, C), -jnp.inf, jnp.float32)
        smin = jnp.full((N, C), jnp.inf, jnp.float32)
        cnt = jnp.zeros((1, K), jnp.float32)
        for m in range(_KNN):
            col = jnp.sum(jnp.where(lane5 == float(m), ids, 0.0), axis=1,
                          keepdims=True)
            mf = (lane == col).astype(jnp.float32)
            g = _dot(mf, pm, ((1,), (1,)))
            ssum = ssum + g
            smax = jnp.maximum(smax, g)
            smin = jnp.minimum(smin, g)
            cnt = cnt + jnp.sum(mf, axis=0, keepdims=True)
        sums.append(ssum)
        mxs.append(smax)
        mns.append(smin)
        s1 = s1 + jnp.sum(ssum, axis=0, keepdims=True) \
            + _KNN * jnp.sum(q, axis=0, keepdims=True)
        s2 = s2 + _dot(cnt, pm * pm, ((1,), (1,))) \
            + 2.0 * jnp.sum(q * ssum, axis=0, keepdims=True) \
            + _KNN * jnp.sum(q * q, axis=0, keepdims=True)

    count = B * N * _KNN
    mean = s1 / count
    var = s2 / count - mean * mean
    a = g1_ref[...] / jnp.sqrt(var + 1e-5)
    bb = b1_ref[...] - mean * a
    for b in range(B):
        meff = jnp.where(a >= 0.0, mxs[b], mns[b])
        y = jnp.maximum(a * (meff + qs[b]) + bb, 0.0)
        out_ref[b] = xn_ref[b] + y


def _run(interpret=False):
    def go(xn, xc, en, w0, g0, b0, anc, sigp, w1a, wd, g1, b1):
        B, N, C = xn.shape
        K = anc.shape[0]
        nodes = pl.pallas_call(
            _stage1_body,
            out_shape=jax.ShapeDtypeStruct((B, K, C), jnp.float32),
            interpret=interpret,
        )(xn, en, w0, g0, b0, anc, sigp)
        m1 = nodes.reshape(B, C, K)

        li = pl.pallas_call(
            _rank_body,
            out_shape=jax.ShapeDtypeStruct((B, _KNN, N), jnp.float32),
            interpret=interpret,
        )(xc, m1)
        ids = li.reshape(B, N, _KNN)

        outn = pl.pallas_call(
            _edgeconv_body,
            out_shape=jax.ShapeDtypeStruct((B, N, C), jnp.float32),
            interpret=interpret,
        )(xn, m1, w1a, wd, ids, g1, b1)
        return outn
    return go


def kernel(x, edge, W0, gamma0, beta0, anchor, sigma_p, W1, gamma1, beta1):
    B, C, H, W = x.shape
    N = H * W
    xc = x.reshape(B, C, N)
    xn = xc.transpose(0, 2, 1)
    en = edge.reshape(B, N, 1)
    w1a = W1[:, :C]
    wd = W1[:, C:] - w1a
    outn = _run()(xn, xc, en, W0, gamma0.reshape(1, C), beta0.reshape(1, C),
                  anchor, sigma_p, w1a, wd,
                  gamma1.reshape(1, C), beta1.reshape(1, C))
    return outn.transpose(0, 2, 1).reshape(B, C, H, W)
